# Initial kernel scaffold; baseline (speedup 1.0000x reference)
import jax
import jax.numpy as jnp
from jax.experimental import pallas as pl


def kernel(x, edge_index, W_value, W_assign, b_assign, W_root, b_root):
    raise NotImplementedError("write your pallas kernel here")


# speedup vs baseline: 8.7572x; 8.7572x over previous
import functools

import jax
import jax.numpy as jnp
from jax import lax
from jax.experimental import pallas as pl
from jax.experimental.pallas import tpu as pltpu
from jax.experimental.pallas import tpu_sc as plsc

N = 10000
IN_CH = 256
OUT_CH = 256
H = 4
E = 160000

BLK = 400
GRID = N // BLK

NPASS = 9
NA = 10240
SA = NA // 16

CH2 = 625
W1 = 32
CH1 = 313
E1 = W1 * CH1 * 16


def _tc1_body(x_ref, wv_ref, wa_ref, p_ref):
    xb = x_ref[...]
    a = jnp.dot(xb, wa_ref[...], preferred_element_type=jnp.float32)
    g = jnp.exp(a)
    vals = jnp.dot(xb, wv_ref[...], preferred_element_type=jnp.float32)
    for p in range(8):
        for h in range(H):
            c0 = h * OUT_CH + p * 32
            p_ref[p, :, h * 32:(h + 1) * 32] = vals[:, c0:c0 + 32] * g[:, h:h + 1]
    p_ref[8, :, 0:H] = g
    p_ref[8, :, H:128] = jnp.zeros((BLK, 128 - H), jnp.float32)


def _tc1(x, wv, wa):
    return pl.pallas_call(
        _tc1_body,
        grid=(GRID,),
        in_specs=[
            pl.BlockSpec((BLK, IN_CH), lambda i: (i, 0)),
            pl.BlockSpec((IN_CH, H * OUT_CH), lambda i: (0, 0)),
            pl.BlockSpec((IN_CH, H), lambda i: (0, 0)),
        ],
        out_specs=pl.BlockSpec((NPASS, BLK, 128), lambda i: (0, i, 0)),
        out_shape=jax.ShapeDtypeStruct((NPASS, N, 128), jnp.float32),
    )(x, wv, wa)


def _tc2_body(acc_ref, x_ref, wr_ref, br_ref, out_ref):
    s = acc_ref[8, :, 0:H] + acc_ref[9, :, 0:H]
    sinv = jnp.where(s > 0.0, 0.25 / s, 0.0)
    xb = x_ref[...]
    root = jnp.dot(xb, wr_ref[...], preferred_element_type=jnp.float32) + br_ref[...]
    cols = []
    for p in range(8):
        accp = acc_ref[p]
        seg = accp[:, 0:32] * sinv[:, 0:1]
        for h in range(1, H):
            seg = seg + accp[:, h * 32:(h + 1) * 32] * sinv[:, h:h + 1]
        cols.append(seg)
    out_ref[...] = jnp.concatenate(cols, axis=1) + root


def _tc2(acc, x, wr, br):
    return pl.pallas_call(
        _tc2_body,
        grid=(GRID,),
        in_specs=[
            pl.BlockSpec((NPASS + 1, BLK, 128), lambda i: (0, i, 0)),
            pl.BlockSpec((BLK, IN_CH), lambda i: (i, 0)),
            pl.BlockSpec((IN_CH, OUT_CH), lambda i: (0, 0)),
            pl.BlockSpec((1, OUT_CH), lambda i: (0, 0)),
        ],
        out_specs=pl.BlockSpec((BLK, OUT_CH), lambda i: (i, 0)),
        out_shape=jax.ShapeDtypeStruct((N, OUT_CH), jnp.float32),
    )(acc, x, wr, br)


_MESH = plsc.VectorSubcoreMesh(core_axis_name="c", subcore_axis_name="s")


@functools.partial(
    pl.kernel,
    mesh=_MESH,
    out_type=jax.ShapeDtypeStruct(((NPASS + 1) * NA, 128), jnp.float32),
    scratch_types=[
        pltpu.VMEM((CH2 * 16,), jnp.int32),
        pltpu.VMEM((CH2 * 16,), jnp.int32),
        pltpu.VMEM((16, 128), jnp.float32),
        pltpu.SemaphoreType.DMA,
        pltpu.VMEM_SHARED((NA, 128), jnp.float32),
    ],
)
def _sc2(p_hbm, row_hbm, col_hbm, rowg_hbm, colg_hbm, z128_hbm,
         acc_hbm, row_v, col_v, rbuf, sem, acc):
    cid = lax.axis_index("c")
    sid = lax.axis_index("s")

    def run_pass(nchunk, gbase, obase):
        pltpu.sync_copy(z128_hbm, acc.at[pl.ds(sid * SA, SA)])
        plsc.subcore_barrier()

        def chunk(j, carry):
            rvec = row_v[pl.ds(j * 16, 16)] + gbase
            cvec = col_v[pl.ds(j * 16, 16)]
            pltpu.async_copy(p_hbm.at[rvec], rbuf, sem).wait()
            pltpu.sync_copy(rbuf, acc.at[cvec], add=True)
            return carry

        lax.fori_loop(0, nchunk, chunk, 0)
        plsc.subcore_barrier()
        pltpu.sync_copy(
            acc.at[pl.ds(sid * SA, SA)],
            acc_hbm.at[pl.ds(obase + sid * SA, SA)],
        )
        plsc.subcore_barrier()

    pltpu.sync_copy(row_hbm.at[pl.ds(sid * (CH2 * 16), CH2 * 16)], row_v)
    pltpu.sync_copy(col_hbm.at[pl.ds(sid * (CH2 * 16), CH2 * 16)], col_v)
    for p in range(4):
        pass_id = cid * 4 + p
        run_pass(CH2, pass_id * N, pass_id * NA)

    wbase = (cid * 16 + sid) * (CH1 * 16)
    pltpu.sync_copy(rowg_hbm.at[pl.ds(wbase, CH1 * 16)],
                    row_v.at[pl.ds(0, CH1 * 16)])
    pltpu.sync_copy(colg_hbm.at[pl.ds(wbase, CH1 * 16)],
                    col_v.at[pl.ds(0, CH1 * 16)])
    run_pass(CH1, 8 * N, (8 + cid) * NA)


@jax.jit
def kernel(x, edge_index, W_value, W_assign, b_assign, W_root, b_root):
    del b_assign
    row = edge_index[0].astype(jnp.int32)
    col = edge_index[1].astype(jnp.int32)

    P = _tc1(x, W_value, W_assign)

    npad = E1 - E
    rowg = jnp.concatenate([row, jnp.zeros((npad,), jnp.int32)])
    colg = jnp.concatenate([col, jnp.full((npad,), N, jnp.int32)])

    accU = _sc2(
        P.reshape(NPASS * N, 128), row, col, rowg, colg,
        jnp.zeros((SA, 128), jnp.float32),
    )

    return _tc2(
        accU.reshape(NPASS + 1, NA, 128), x, W_root, b_root.reshape(1, OUT_CH)
    )


# speedup vs baseline: 27.6421x; 3.1565x over previous
import functools

import jax
import jax.numpy as jnp
from jax import lax
from jax.experimental import pallas as pl
from jax.experimental.pallas import tpu as pltpu
from jax.experimental.pallas import tpu_sc as plsc

N = 10000
IN_CH = 256
OUT_CH = 256
H = 4
E = 160000

BLK = 400
GRID = N // BLK

NPASS = 9
NA = 10240
SA = NA // 16

K = 80
ET = E // 16
CHF = ET // K
ETG = 5040
EG = 32 * ETG
CHG = ETG // K


def _tc1_body(x_ref, wv_ref, wa_ref, p_ref):
    xb = x_ref[...]
    a = jnp.dot(xb, wa_ref[...], preferred_element_type=jnp.float32)
    g = jnp.exp(a)
    vals = jnp.dot(xb, wv_ref[...], preferred_element_type=jnp.float32)
    for p in range(8):
        for h in range(H):
            c0 = h * OUT_CH + p * 32
            p_ref[p, :, h * 32:(h + 1) * 32] = vals[:, c0:c0 + 32] * g[:, h:h + 1]
    p_ref[8, :, 0:H] = g
    p_ref[8, :, H:128] = jnp.zeros((BLK, 128 - H), jnp.float32)


def _tc1(x, wv, wa):
    return pl.pallas_call(
        _tc1_body,
        grid=(GRID,),
        in_specs=[
            pl.BlockSpec((BLK, IN_CH), lambda i: (i, 0)),
            pl.BlockSpec((IN_CH, H * OUT_CH), lambda i: (0, 0)),
            pl.BlockSpec((IN_CH, H), lambda i: (0, 0)),
        ],
        out_specs=pl.BlockSpec((NPASS, BLK, 128), lambda i: (0, i, 0)),
        out_shape=jax.ShapeDtypeStruct((NPASS, N, 128), jnp.float32),
    )(x, wv, wa)


def _tc2_body(acc_ref, x_ref, wr_ref, br_ref, out_ref):
    s = acc_ref[8, :, 0:H] + acc_ref[9, :, 0:H]
    sinv = jnp.where(s > 0.0, 0.25 / s, 0.0)
    xb = x_ref[...]
    root = jnp.dot(xb, wr_ref[...], preferred_element_type=jnp.float32) + br_ref[...]
    cols = []
    for p in range(8):
        accp = acc_ref[p]
        seg = accp[:, 0:32] * sinv[:, 0:1]
        for h in range(1, H):
            seg = seg + accp[:, h * 32:(h + 1) * 32] * sinv[:, h:h + 1]
        cols.append(seg)
    out_ref[...] = jnp.concatenate(cols, axis=1) + root


def _tc2(acc, x, wr, br):
    return pl.pallas_call(
        _tc2_body,
        grid=(GRID,),
        in_specs=[
            pl.BlockSpec((NPASS + 1, BLK, 128), lambda i: (0, i, 0)),
            pl.BlockSpec((BLK, IN_CH), lambda i: (i, 0)),
            pl.BlockSpec((IN_CH, OUT_CH), lambda i: (0, 0)),
            pl.BlockSpec((1, OUT_CH), lambda i: (0, 0)),
        ],
        out_specs=pl.BlockSpec((BLK, OUT_CH), lambda i: (i, 0)),
        out_shape=jax.ShapeDtypeStruct((N, OUT_CH), jnp.float32),
    )(acc, x, wr, br)


_MESH = plsc.VectorSubcoreMesh(core_axis_name="c", subcore_axis_name="s")


@functools.partial(
    pl.kernel,
    mesh=_MESH,
    out_type=jax.ShapeDtypeStruct(((NPASS + 1) * NA, 128), jnp.float32),
    scratch_types=[
        pltpu.VMEM((ET,), jnp.int32),
        pltpu.VMEM((ET,), jnp.int32),
        pltpu.VMEM((K, 128), jnp.float32),
        pltpu.VMEM((K, 128), jnp.float32),
        pltpu.SemaphoreType.DMA,
        pltpu.SemaphoreType.DMA,
        pltpu.VMEM_SHARED((NA, 128), jnp.float32),
    ],
)
def _sc2(p_hbm, rowsf_hbm, col_hbm, rowsg_hbm, colg_hbm, z128_hbm,
         acc_hbm, row_v, col_v, buf0, buf1, sem0, sem1, acc):
    cid = lax.axis_index("c")
    sid = lax.axis_index("s")

    def gather(c, buf, sem):
        pltpu.async_copy(p_hbm.at[row_v.at[pl.ds(c * K, K)]], buf, sem)

    def gwait(buf, sem):
        pltpu.make_async_copy(p_hbm.at[row_v.at[pl.ds(0, K)]], buf, sem).wait()

    def scatter(c, buf):
        for i in range(K // 16):
            cvec = col_v[pl.ds(c * K + i * 16, 16)]
            pltpu.sync_copy(buf.at[pl.ds(i * 16, 16)], acc.at[cvec], add=True)

    def run_pass(nchunk, obase):
        pltpu.sync_copy(z128_hbm, acc.at[pl.ds(sid * SA, SA)])
        plsc.subcore_barrier()
        gather(0, buf0, sem0)
        gather(1, buf1, sem1)

        def pair(j, carry):
            c0 = 2 * j
            gwait(buf0, sem0)
            scatter(c0, buf0)
            gather(c0 + 2, buf0, sem0)
            gwait(buf1, sem1)
            scatter(c0 + 1, buf1)

            @pl.when(c0 + 3 < nchunk)
            def _():
                gather(c0 + 3, buf1, sem1)

            return carry

        lax.fori_loop(0, (nchunk - 1) // 2, pair, 0)
        gwait(buf0, sem0)
        scatter(nchunk - 1, buf0)
        plsc.subcore_barrier()
        pltpu.sync_copy(
            acc.at[pl.ds(sid * SA, SA)],
            acc_hbm.at[pl.ds(obase + sid * SA, SA)],
        )
        plsc.subcore_barrier()

    pltpu.sync_copy(col_hbm.at[pl.ds(sid * ET, ET)], col_v)
    for p in range(4):
        pass_id = cid * 4 + p
        pltpu.sync_copy(
            rowsf_hbm.at[pl.ds(pass_id * E + sid * ET, ET)], row_v
        )
        run_pass(CHF, pass_id * NA)

    wbase = (cid * 16 + sid) * ETG
    pltpu.sync_copy(rowsg_hbm.at[pl.ds(wbase, ETG)], row_v.at[pl.ds(0, ETG)])
    pltpu.sync_copy(colg_hbm.at[pl.ds(wbase, ETG)], col_v.at[pl.ds(0, ETG)])
    run_pass(CHG, (8 + cid) * NA)


@jax.jit
def kernel(x, edge_index, W_value, W_assign, b_assign, W_root, b_root):
    del b_assign
    row = edge_index[0].astype(jnp.int32)
    col = edge_index[1].astype(jnp.int32)

    P = _tc1(x, W_value, W_assign)

    rowsf = (row[None, :] + (jnp.arange(8, dtype=jnp.int32) * N)[:, None]).reshape(-1)
    npad = EG - E
    rowsg = jnp.concatenate(
        [row + 8 * N, jnp.full((npad,), 8 * N, jnp.int32)]
    )
    colg = jnp.concatenate([col, jnp.full((npad,), N, jnp.int32)])

    accU = _sc2(
        P.reshape(NPASS * N, 128), rowsf, col, rowsg, colg,
        jnp.zeros((SA, 128), jnp.float32),
    )

    return _tc2(
        accU.reshape(NPASS + 1, NA, 128), x, W_root, b_root.reshape(1, OUT_CH)
    )
